# baseline (speedup 1.0000x reference)
import jax
import jax.numpy as jnp
from jax.experimental import pallas as pl
from jax.experimental.pallas import tpu as pltpu

_B = 256
_D = 256
_STEPS = 4
_TILE = 4096
_HI = jax.lax.Precision.HIGHEST


_DN_T = (((0,), (0,)), ((), ()))


def _split_dot(a_bf16, b_f32, dn=None):
    b_hi = b_f32.astype(jnp.bfloat16)
    b_lo = (b_f32 - b_hi.astype(jnp.float32)).astype(jnp.bfloat16)
    if dn is None:
        hi = jax.lax.dot(a_bf16, b_hi, preferred_element_type=jnp.float32)
        lo = jax.lax.dot(a_bf16, b_lo, preferred_element_type=jnp.float32)
    else:
        hi = jax.lax.dot_general(a_bf16, b_hi, dn,
                                 preferred_element_type=jnp.float32)
        lo = jax.lax.dot_general(a_bf16, b_lo, dn,
                                 preferred_element_type=jnp.float32)
    return hi + lo


def _body(x_ref, b_ref, wih_ref, whh_ref, bias_ref, out_ref,
          qstar, h, c, q, m, s_acc, r_acc):
    st = pl.program_id(0)
    t = pl.program_id(1)
    num_tiles = pl.num_programs(1)

    @pl.when(jnp.logical_and(st == 0, t == 0))
    def _init():
        qstar[...] = jnp.zeros_like(qstar)
        h[...] = jnp.zeros_like(h)
        c[...] = jnp.zeros_like(c)

    @pl.when(t == 0)
    def _lstm():
        gates = (jnp.dot(qstar[...], wih_ref[...], precision=_HI)
                 + jnp.dot(h[...], whh_ref[...], precision=_HI)
                 + bias_ref[...])
        i = jax.nn.sigmoid(gates[:, 0:_D])
        f = jax.nn.sigmoid(gates[:, _D:2 * _D])
        g = jnp.tanh(gates[:, 2 * _D:3 * _D])
        o = jax.nn.sigmoid(gates[:, 3 * _D:4 * _D])
        cc = f * c[...] + i * g
        c[...] = cc
        hh = o * jnp.tanh(cc)
        h[...] = hh
        q[...] = hh
        m[...] = jnp.full_like(m, -jnp.inf)
        s_acc[...] = jnp.zeros_like(s_acc)
        r_acc[...] = jnp.zeros_like(r_acc)

    ids = b_ref[0, 0, :]
    cols = jax.lax.broadcasted_iota(jnp.int32, (_TILE, _B), 1)
    onehot_b = (ids[:, None] == cols)
    onehot_bf = onehot_b.astype(jnp.bfloat16)

    qg = _split_dot(onehot_bf, q[...])
    x = x_ref[...]
    e = jnp.sum(x * qg, axis=1)

    m_old = m[0, :]
    tile_max = jnp.max(jnp.where(onehot_b, e[:, None], -jnp.inf), axis=0)
    m_new = jnp.maximum(m_old, tile_max)
    m[0, :] = m_new
    scale = jnp.where(m_new == -jnp.inf, 1.0, jnp.exp(m_old - m_new))

    mg = jnp.sum(jnp.where(onehot_b, m_new[None, :], 0.0), axis=1)
    ee = jnp.exp(e - mg)
    s_acc[0, :] = (s_acc[0, :] * scale
                   + jnp.sum(jnp.where(onehot_b, ee[:, None], 0.0), axis=0))
    wx = (ee[:, None] * x).astype(jnp.bfloat16)
    r_acc[...] = (r_acc[...] * scale[:, None]
                  + jax.lax.dot_general(onehot_bf, wx, _DN_T,
                                        preferred_element_type=jnp.float32))

    @pl.when(t == num_tiles - 1)
    def _finish():
        r = r_acc[...] / (s_acc[0, :][:, None] + 1e-16)
        qstar[:, 0:_D] = q[...]
        qstar[:, _D:2 * _D] = r

        @pl.when(st == _STEPS - 1)
        def _out():
            out_ref[...] = qstar[...]


def _set2set_tc(x, batch_i32, w_iht, w_hht, bias):
    n = x.shape[0]
    num_tiles = pl.cdiv(n, _TILE)
    n_pad = num_tiles * _TILE
    x_pad = jnp.pad(x, ((0, n_pad - n), (0, 0)))
    b_pad = jnp.pad(batch_i32, (0, n_pad - n), constant_values=_B)
    b3 = b_pad.reshape(num_tiles, 1, _TILE)

    return pl.pallas_call(
        _body,
        grid=(_STEPS, num_tiles),
        in_specs=[
            pl.BlockSpec((_TILE, _D), lambda s, t: (t, 0)),
            pl.BlockSpec((1, 1, _TILE), lambda s, t: (t, 0, 0)),
            pl.BlockSpec((2 * _D, 4 * _D), lambda s, t: (0, 0)),
            pl.BlockSpec((_D, 4 * _D), lambda s, t: (0, 0)),
            pl.BlockSpec((1, 4 * _D), lambda s, t: (0, 0)),
        ],
        out_specs=pl.BlockSpec((_B, 2 * _D), lambda s, t: (0, 0)),
        out_shape=jax.ShapeDtypeStruct((_B, 2 * _D), jnp.float32),
        scratch_shapes=[
            pltpu.VMEM((_B, 2 * _D), jnp.float32),
            pltpu.VMEM((_B, _D), jnp.float32),
            pltpu.VMEM((_B, _D), jnp.float32),
            pltpu.VMEM((_B, _D), jnp.float32),
            pltpu.VMEM((1, _B), jnp.float32),
            pltpu.VMEM((1, _B), jnp.float32),
            pltpu.VMEM((_B, _D), jnp.float32),
        ],
        compiler_params=pltpu.CompilerParams(
            dimension_semantics=("arbitrary", "arbitrary")),
    )(x_pad, b3, w_iht, w_hht, bias)


def kernel(x, batch, W_ih, W_hh, b_ih, b_hh):
    batch = batch.astype(jnp.int32)
    bias = (b_ih + b_hh).reshape(1, 4 * _D).astype(jnp.float32)
    return _set2set_tc(x, batch, W_ih.T, W_hh.T, bias)
